# baseline (speedup 1.0000x reference)
import functools

import jax
import jax.numpy as jnp
from jax import lax
from jax.experimental import pallas as pl
from jax.experimental.pallas import tpu as pltpu
from jax.experimental.pallas import tpu_sc as plsc

_B, _N1, _N2, _C2, _MLP = 16, 1024, 256, 512, 256


def _knn_kernel(xyz1_ref, feat1_ref, xyz2t_ref, W1b_ref, b1_ref,
                idx_ref, wv_ref, h1_ref, *, n2):
    b = pl.program_id(0)
    x1 = xyz1_ref[0]
    x2t = xyz2t_ref[0]
    blk = x1.shape[0]

    d = jnp.zeros((blk, n2), dtype=jnp.float32)
    for k in range(3):
        diff = x1[:, k:k + 1] - x2t[k:k + 1, :]
        d = d + diff * diff

    inf = jnp.float32(jnp.inf)
    lane = jax.lax.broadcasted_iota(jnp.int32, (blk, n2), 1)
    m1 = jnp.min(d, axis=1, keepdims=True)
    i1 = jnp.min(jnp.where(d == m1, lane, n2), axis=1, keepdims=True)
    k2 = jnp.where(lane == i1, inf, d)
    m2 = jnp.min(k2, axis=1, keepdims=True)
    i2 = jnp.min(jnp.where(k2 == m2, lane, n2), axis=1, keepdims=True)
    k3 = jnp.where(lane == i2, inf, k2)
    m3 = jnp.min(k3, axis=1, keepdims=True)
    i3 = jnp.min(jnp.where(k3 == m3, lane, n2), axis=1, keepdims=True)

    w1 = 1.0 / jnp.maximum(m1, 1e-10)
    w2 = 1.0 / jnp.maximum(m2, 1e-10)
    w3 = 1.0 / jnp.maximum(m3, 1e-10)
    inv = 1.0 / (w1 + w2 + w3)

    base = b * n2
    for k, (ik, wk) in enumerate(((i1, w1), (i2, w2), (i3, w3))):
        idx_ref[0, k, :] = (ik + base)[:, 0]
        wv_ref[0, k, :, :] = jnp.broadcast_to(wk * inv, (blk, 16))

    h1_ref[0] = jnp.dot(feat1_ref[0], W1b_ref[...],
                        preferred_element_type=jnp.float32) + b1_ref[...]


def _mlp_kernel(interp_ref, h1_ref, W1a_ref, W2_ref, b2_ref, out_ref):
    h = h1_ref[0] + jnp.dot(interp_ref[0], W1a_ref[...],
                            preferred_element_type=jnp.float32)
    h = jnp.maximum(h, 0.0)
    out = jnp.dot(h, W2_ref[...], preferred_element_type=jnp.float32)
    out_ref[0] = jnp.maximum(out + b2_ref[...], 0.0)


_NW = 32
_CH = 32
_Q = _B * _N1
_QW = _Q // _NW


def _sc_interp(table, idxf, wv):
    mesh = plsc.VectorSubcoreMesh(core_axis_name="c", subcore_axis_name="s")

    @functools.partial(
        pl.kernel, mesh=mesh,
        out_type=jax.ShapeDtypeStruct((_Q, _C2), jnp.float32),
        scratch_types=[
            pltpu.VMEM((_CH,), jnp.int32),
            pltpu.VMEM((_CH,), jnp.int32),
            pltpu.VMEM((_CH,), jnp.int32),
            pltpu.VMEM((_CH, _C2), jnp.float32),
            pltpu.VMEM((_CH, _C2), jnp.float32),
            pltpu.VMEM((_CH, _C2), jnp.float32),
            pltpu.VMEM((3, _CH, 16), jnp.float32),
            pltpu.VMEM((_CH, _C2), jnp.float32),
            pltpu.SemaphoreType.DMA,
            pltpu.SemaphoreType.DMA,
            pltpu.SemaphoreType.DMA,
        ],
    )
    def k(table_hbm, idx_hbm, wv_hbm, out_hbm,
          i0_v, i1_v, i2_v, r0_v, r1_v, r2_v, w_v, out_v, s0, s1, s2):
        wid = lax.axis_index("s") * 2 + lax.axis_index("c")
        qw_base = wid * _QW
        bidx = qw_base // _N1

        def chunk_body(c, _):
            q0 = qw_base + c * _CH
            n0 = q0 - bidx * _N1
            pltpu.sync_copy(idx_hbm.at[bidx, 0, pl.ds(n0, _CH)], i0_v)
            pltpu.sync_copy(idx_hbm.at[bidx, 1, pl.ds(n0, _CH)], i1_v)
            pltpu.sync_copy(idx_hbm.at[bidx, 2, pl.ds(n0, _CH)], i2_v)
            pltpu.sync_copy(wv_hbm.at[bidx, :, pl.ds(n0, _CH), :], w_v)
            cp0 = pltpu.async_copy(table_hbm.at[i0_v], r0_v, s0)
            cp1 = pltpu.async_copy(table_hbm.at[i1_v], r1_v, s1)
            cp2 = pltpu.async_copy(table_hbm.at[i2_v], r2_v, s2)
            cp0.wait()
            cp1.wait()
            cp2.wait()

            def q_body(q, _):
                w0 = w_v[0, q, :]
                w1 = w_v[1, q, :]
                w2 = w_v[2, q, :]
                for l in range(_C2 // 16):
                    sl = pl.ds(l * 16, 16)
                    acc = w0 * r0_v[q, sl]
                    acc = acc + w1 * r1_v[q, sl]
                    acc = acc + w2 * r2_v[q, sl]
                    out_v[q, sl] = acc
                return 0

            lax.fori_loop(0, _CH, q_body, 0)
            pltpu.sync_copy(out_v, out_hbm.at[pl.ds(q0, _CH)])
            return 0

        lax.fori_loop(0, _QW // _CH, chunk_body, 0)

    return k(table, idxf, wv)


@jax.jit
def kernel(xyz1, feat1, xyz2, feat2, W1, b1, W2, b2):
    xyz2t = jnp.swapaxes(xyz2, 1, 2)
    b1r = b1.reshape(1, _MLP)
    b2r = b2.reshape(1, _MLP)
    W1a = W1[:_C2]
    W1b = W1[_C2:]

    idxf, wv, h1 = pl.pallas_call(
        functools.partial(_knn_kernel, n2=_N2),
        grid=(_B,),
        in_specs=[
            pl.BlockSpec((1, _N1, 3), lambda b: (b, 0, 0)),
            pl.BlockSpec((1, _N1, 256), lambda b: (b, 0, 0)),
            pl.BlockSpec((1, 3, _N2), lambda b: (b, 0, 0)),
            pl.BlockSpec((256, _MLP), lambda b: (0, 0)),
            pl.BlockSpec((1, _MLP), lambda b: (0, 0)),
        ],
        out_specs=[
            pl.BlockSpec((1, 3, _N1), lambda b: (b, 0, 0)),
            pl.BlockSpec((1, 3, _N1, 16), lambda b: (b, 0, 0, 0)),
            pl.BlockSpec((1, _N1, _MLP), lambda b: (b, 0, 0)),
        ],
        out_shape=[
            jax.ShapeDtypeStruct((_B, 3, _N1), jnp.int32),
            jax.ShapeDtypeStruct((_B, 3, _N1, 16), jnp.float32),
            jax.ShapeDtypeStruct((_B, _N1, _MLP), jnp.float32),
        ],
    )(xyz1, feat1, xyz2t, W1b, b1r)

    table = feat2.reshape(_B * _N2, _C2)
    interp = _sc_interp(table, idxf, wv).reshape(_B, _N1, _C2)

    out = pl.pallas_call(
        _mlp_kernel,
        grid=(_B,),
        in_specs=[
            pl.BlockSpec((1, _N1, _C2), lambda b: (b, 0, 0)),
            pl.BlockSpec((1, _N1, _MLP), lambda b: (b, 0, 0)),
            pl.BlockSpec((_C2, _MLP), lambda b: (0, 0)),
            pl.BlockSpec((_MLP, _MLP), lambda b: (0, 0)),
            pl.BlockSpec((1, _MLP), lambda b: (0, 0)),
        ],
        out_specs=pl.BlockSpec((1, _N1, _MLP), lambda b: (b, 0, 0)),
        out_shape=jax.ShapeDtypeStruct((_B, _N1, _MLP), jnp.float32),
    )(interp, h1, W1a, W2, b2r)
    return out


# speedup vs baseline: 5.2243x; 5.2243x over previous
import functools

import jax
import jax.numpy as jnp
from jax.experimental import pallas as pl
from jax.experimental.pallas import tpu as pltpu


def _fp_kernel(xyz1_ref, feat1_ref, xyz2t_ref, feat2_ref, W1_ref, b1_ref,
               W2_ref, b2_ref, out_ref, S_scr, G_scr, *, n2, c2):
    t = pl.program_id(0)
    cur = t % 2
    prev = (t + 1) % 2

    x1 = xyz1_ref[0]
    x2t = xyz2t_ref[0]
    blk = x1.shape[0]

    d = jnp.zeros((blk, n2), dtype=jnp.float32)
    for k in range(3):
        diff = x1[:, k:k + 1] - x2t[k:k + 1, :]
        d = d + diff * diff

    inf = jnp.float32(jnp.inf)
    m1 = jnp.min(d, axis=1, keepdims=True)
    k2 = jnp.where(d == m1, inf, d)
    m2 = jnp.min(k2, axis=1, keepdims=True)
    k3 = jnp.where(k2 == m2, inf, k2)
    m3 = jnp.min(k3, axis=1, keepdims=True)

    nn_mask = d <= m3
    w = jnp.where(nn_mask, 1.0 / jnp.maximum(d, 1e-10), 0.0)
    denom = jnp.sum(w, axis=1, keepdims=True)
    S_scr[cur] = w * (1.0 / denom)
    G_scr[cur] = jnp.dot(feat2_ref[0], W1_ref[:c2, :],
                         preferred_element_type=jnp.float32)

    h = jnp.dot(S_scr[prev], G_scr[prev], preferred_element_type=jnp.float32)
    h = h + jnp.dot(feat1_ref[0], W1_ref[c2:, :],
                    preferred_element_type=jnp.float32)
    h = jnp.maximum(h + b1_ref[...], 0.0)
    out = jnp.dot(h, W2_ref[...], preferred_element_type=jnp.float32)
    out_ref[0] = jnp.maximum(out + b2_ref[...], 0.0)


@jax.jit
def kernel(xyz1, feat1, xyz2, feat2, W1, b1, W2, b2):
    B, N1, _ = xyz1.shape
    _, N2, C2 = feat2.shape
    C1 = feat1.shape[-1]
    MLP = W2.shape[-1]

    xyz2t = jnp.swapaxes(xyz2, 1, 2)
    b1r = b1.reshape(1, MLP)
    b2r = b2.reshape(1, MLP)

    def sel_ix(t):
        return (jnp.minimum(t, B - 1), 0, 0)

    def mm_ix(t):
        return (jnp.maximum(t - 1, 0), 0, 0)

    out = pl.pallas_call(
        functools.partial(_fp_kernel, n2=N2, c2=C2),
        grid=(B + 1,),
        in_specs=[
            pl.BlockSpec((1, N1, 3), sel_ix),
            pl.BlockSpec((1, N1, C1), mm_ix),
            pl.BlockSpec((1, 3, N2), sel_ix),
            pl.BlockSpec((1, N2, C2), sel_ix),
            pl.BlockSpec((C1 + C2, MLP), lambda t: (0, 0)),
            pl.BlockSpec((1, MLP), lambda t: (0, 0)),
            pl.BlockSpec((MLP, MLP), lambda t: (0, 0)),
            pl.BlockSpec((1, MLP), lambda t: (0, 0)),
        ],
        out_specs=pl.BlockSpec((1, N1, MLP), mm_ix),
        out_shape=jax.ShapeDtypeStruct((B, N1, MLP), jnp.float32),
        scratch_shapes=[
            pltpu.VMEM((2, N1, N2), jnp.float32),
            pltpu.VMEM((2, N2, MLP), jnp.float32),
        ],
    )(xyz1, feat1, xyz2t, feat2, W1, b1r, W2, b2r)
    return out


# speedup vs baseline: 5.5232x; 1.0572x over previous
import functools

import jax
import jax.numpy as jnp
from jax.experimental import pallas as pl
from jax.experimental.pallas import tpu as pltpu

_BLK_N1 = 1024


def _fp_kernel(xyz1_ref, feat1_ref, xyz2t_ref, feat2_ref, W1_ref, b1_ref,
               W2_ref, b2_ref, out_ref, G_scr, *, n2, c2):
    i = pl.program_id(1)

    @pl.when(i == 0)
    def _compute_g():
        G_scr[...] = jnp.dot(feat2_ref[0], W1_ref[:c2, :],
                             preferred_element_type=jnp.float32)

    x1 = xyz1_ref[0]
    x2t = xyz2t_ref[0]
    blk = x1.shape[0]

    d = jnp.zeros((blk, n2), dtype=jnp.float32)
    for k in range(3):
        diff = x1[:, k:k + 1] - x2t[k:k + 1, :]
        d = d + diff * diff

    inf = jnp.float32(jnp.inf)
    m1 = jnp.min(d, axis=1, keepdims=True)
    k2 = jnp.where(d == m1, inf, d)
    m2 = jnp.min(k2, axis=1, keepdims=True)
    k3 = jnp.where(k2 == m2, inf, k2)
    m3 = jnp.min(k3, axis=1, keepdims=True)

    nn_mask = d <= m3
    w = jnp.where(nn_mask, 1.0 / jnp.maximum(d, 1e-10), 0.0)
    denom = jnp.sum(w, axis=1, keepdims=True)
    S = w * (1.0 / denom)

    h = jnp.dot(S, G_scr[...], preferred_element_type=jnp.float32)
    h = h + jnp.dot(feat1_ref[0], W1_ref[c2:, :],
                    preferred_element_type=jnp.float32)
    h = jnp.maximum(h + b1_ref[...], 0.0)
    out = jnp.dot(h, W2_ref[...], preferred_element_type=jnp.float32)
    out_ref[0] = jnp.maximum(out + b2_ref[...], 0.0)


@jax.jit
def kernel(xyz1, feat1, xyz2, feat2, W1, b1, W2, b2):
    B, N1, _ = xyz1.shape
    _, N2, C2 = feat2.shape
    C1 = feat1.shape[-1]
    MLP = W2.shape[-1]
    blk = _BLK_N1
    nb = N1 // blk

    xyz2t = jnp.swapaxes(xyz2, 1, 2)
    b1r = b1.reshape(1, MLP)
    b2r = b2.reshape(1, MLP)

    grid = (B, nb)
    out = pl.pallas_call(
        functools.partial(_fp_kernel, n2=N2, c2=C2),
        grid=grid,
        in_specs=[
            pl.BlockSpec((1, blk, 3), lambda b, i: (b, i, 0)),
            pl.BlockSpec((1, blk, C1), lambda b, i: (b, i, 0)),
            pl.BlockSpec((1, 3, N2), lambda b, i: (b, 0, 0)),
            pl.BlockSpec((1, N2, C2), lambda b, i: (b, 0, 0)),
            pl.BlockSpec((C1 + C2, MLP), lambda b, i: (0, 0)),
            pl.BlockSpec((1, MLP), lambda b, i: (0, 0)),
            pl.BlockSpec((MLP, MLP), lambda b, i: (0, 0)),
            pl.BlockSpec((1, MLP), lambda b, i: (0, 0)),
        ],
        out_specs=pl.BlockSpec((1, blk, MLP), lambda b, i: (b, i, 0)),
        out_shape=jax.ShapeDtypeStruct((B, N1, MLP), jnp.float32),
        scratch_shapes=[pltpu.VMEM((N2, MLP), jnp.float32)],
    )(xyz1, feat1, xyz2t, feat2, W1, b1r, W2, b2r)
    return out
